# baseline (speedup 1.0000x reference)
import functools

import jax
import jax.numpy as jnp
from jax import lax
from jax.experimental import pallas as pl
from jax.experimental.pallas import tpu as pltpu
from jax.experimental.pallas import tpu_sc as plsc

B = 8
Q = 20000
T = 100
TPAD = 112
NCORE = 2
NSUB = 16
B_PER_CORE = B // NCORE
G = 4

QS = 6688
QT = Q - QS
NT = QT // 128
TG = 13
PER_W = 432
CHUNKS = PER_W // 16
BIG = 1 << 30


def _sc_body(pred_hbm, tgt_hbm, outmax_hbm, outidx_hbm,
             praw, x0a, y0a, x1a, y1a, aa, traw,
             lmax, lidx, mgmax, mgidx, omax, oidx,
             shmax, shidx):
    c = lax.axis_index("c")
    s = lax.axis_index("s")
    iota = lax.iota(jnp.int32, 16)
    lane0 = iota == 0
    zc = jnp.zeros((16,), jnp.int32)
    base = jnp.minimum(s * PER_W, QS - PER_W)

    for bl in range(B_PER_CORE):
        b = c * B_PER_CORE + bl
        pltpu.sync_copy(pred_hbm.at[pl.ds(b * (Q * 4) + base * 4, PER_W * 4)], praw)
        pltpu.sync_copy(tgt_hbm.at[pl.ds(b * (T * 4), T * 4)], traw.at[pl.ds(0, T * 4)])

        def pre(j, _):
            r = j * 64 + iota * 4
            cx = plsc.load_gather(praw, [r])
            cy = plsc.load_gather(praw, [r + 1])
            w = plsc.load_gather(praw, [r + 2])
            h = plsc.load_gather(praw, [r + 3])
            x0 = cx - 0.5 * w
            y0 = cy - 0.5 * h
            x1 = cx + 0.5 * w
            y1 = cy + 0.5 * h
            sl = pl.ds(j * 16, 16)
            x0a[sl] = x0
            y0a[sl] = y0
            x1a[sl] = x1
            y1a[sl] = y1
            aa[sl] = (x1 - x0) * (y1 - y0)
            return 0

        lax.fori_loop(0, CHUNKS, pre, 0)

        def per_g(g, _):
            t0 = g * G
            tco = []
            for i in range(G):
                t4 = zc + (t0 + i) * 4
                tcx = plsc.load_gather(traw, [t4])
                tcy = plsc.load_gather(traw, [t4 + 1])
                tw = plsc.load_gather(traw, [t4 + 2])
                th = plsc.load_gather(traw, [t4 + 3])
                tx0 = tcx - 0.5 * tw
                ty0 = tcy - 0.5 * th
                tx1 = tcx + 0.5 * tw
                ty1 = tcy + 0.5 * th
                ta = (tx1 - tx0) * (ty1 - ty0)
                tco.append((tx0, ty0, tx1, ty1, ta))

            def scan_k(k, carry):
                ms, bis, idxv = carry
                sl = pl.ds(k * 16, 16)
                x0 = x0a[sl]
                y0 = y0a[sl]
                x1 = x1a[sl]
                y1 = y1a[sl]
                av = aa[sl]
                nms, nbis = [], []
                for i in range(G):
                    tx0, ty0, tx1, ty1, ta = tco[i]
                    ltx = jnp.maximum(x0, tx0)
                    lty = jnp.maximum(y0, ty0)
                    rbx = jnp.minimum(x1, tx1)
                    rby = jnp.minimum(y1, ty1)
                    inter = jnp.maximum(rbx - ltx, 0.0) * jnp.maximum(rby - lty, 0.0)
                    union = av + ta - inter
                    iou = inter / jnp.maximum(union, 1e-9)
                    lcx = jnp.minimum(x0, tx0)
                    lcy = jnp.minimum(y0, ty0)
                    rcx = jnp.maximum(x1, tx1)
                    rcy = jnp.maximum(y1, ty1)
                    areac = jnp.maximum(rcx - lcx, 0.0) * jnp.maximum(rcy - lcy, 0.0)
                    gv = iou - (areac - union) / jnp.maximum(areac, 1e-9)
                    upd = gv > ms[i]
                    nms.append(jnp.where(upd, gv, ms[i]))
                    nbis.append(jnp.where(upd, idxv, bis[i]))
                return tuple(nms), tuple(nbis), idxv + 16

            m0 = jnp.full((16,), -3.0e38, jnp.float32)
            bi0 = jnp.zeros((16,), jnp.int32)
            ms, bis, _ = lax.fori_loop(
                0, CHUNKS, scan_k,
                ((m0,) * G, (bi0,) * G, base + iota))
            for i in range(G):
                gm = jnp.max(ms[i])
                cand = jnp.where(ms[i] == jnp.full((16,), gm),
                                 bis[i], jnp.full((16,), BIG, jnp.int32))
                gi = jnp.min(cand)
                posv = zc + (bl * TPAD + t0 + i)
                plsc.store_scatter(lmax, [posv], jnp.full((16,), gm), mask=lane0)
                plsc.store_scatter(lidx, [posv], jnp.full((16,), gi, jnp.int32),
                                   mask=lane0)
            return 0

        lax.fori_loop(0, T // G, per_g, 0)

    pltpu.sync_copy(lmax, shmax.at[pl.ds(s * (B_PER_CORE * TPAD), B_PER_CORE * TPAD)])
    pltpu.sync_copy(lidx, shidx.at[pl.ds(s * (B_PER_CORE * TPAD), B_PER_CORE * TPAD)])
    plsc.subcore_barrier()

    @pl.when(s < B_PER_CORE)
    def _merge():
        for w in range(NSUB):
            pltpu.sync_copy(shmax.at[pl.ds(w * (B_PER_CORE * TPAD) + s * TPAD, TPAD)],
                            mgmax.at[pl.ds(w * TPAD, TPAD)])
            pltpu.sync_copy(shidx.at[pl.ds(w * (B_PER_CORE * TPAD) + s * TPAD, TPAD)],
                            mgidx.at[pl.ds(w * TPAD, TPAD)])

        def mg(cc, _):
            sl = pl.ds(cc * 16, 16)
            del _
            acc = mgmax[sl]
            acci = mgidx[sl]
            for w in range(1, NSUB):
                wsl = pl.ds(w * TPAD + cc * 16, 16)
                v = mgmax[wsl]
                vi = mgidx[wsl]
                upd = (v > acc) | ((v == acc) & (vi < acci))
                acc = jnp.where(upd, v, acc)
                acci = jnp.where(upd, vi, acci)
            omax[sl] = acc
            oidx[sl] = acci
            return 0

        lax.fori_loop(0, TPAD // 16, mg, 0)
        gb = c * B_PER_CORE + s
        pltpu.sync_copy(omax, outmax_hbm.at[pl.ds(gb * TPAD, TPAD)])
        pltpu.sync_copy(oidx, outidx_hbm.at[pl.ds(gb * TPAD, TPAD)])


def _tc_body(pref, tref, omax_ref, oidx_ref, x0s, y0s, x1s, y1s, aas):
    g = pl.program_id(1)

    @pl.when(g == 0)
    def _pre():
        cx = pref[0, 0]
        cy = pref[0, 1]
        w = pref[0, 2]
        h = pref[0, 3]
        x0 = cx - 0.5 * w
        y0 = cy - 0.5 * h
        x1 = cx + 0.5 * w
        y1 = cy + 0.5 * h
        x0s[...] = x0
        y0s[...] = y0
        x1s[...] = x1
        y1s[...] = y1
        aas[...] = (x1 - x0) * (y1 - y0)

    tcx = tref[0, pl.ds(g * 8, 8), 0:1]
    tcy = tref[0, pl.ds(g * 8, 8), 1:2]
    tw = tref[0, pl.ds(g * 8, 8), 2:3]
    th = tref[0, pl.ds(g * 8, 8), 3:4]
    tx0 = tcx - 0.5 * tw
    ty0 = tcy - 0.5 * th
    tx1 = tcx + 0.5 * tw
    ty1 = tcy + 0.5 * th
    ta = (tx1 - tx0) * (ty1 - ty0)
    iota = lax.broadcasted_iota(jnp.int32, (1, 128), 1)

    def body(k, carry):
        m, bi = carry
        x0 = x0s[pl.ds(k, 1), :]
        y0 = y0s[pl.ds(k, 1), :]
        x1 = x1s[pl.ds(k, 1), :]
        y1 = y1s[pl.ds(k, 1), :]
        av = aas[pl.ds(k, 1), :]
        ltx = jnp.maximum(x0, tx0)
        lty = jnp.maximum(y0, ty0)
        rbx = jnp.minimum(x1, tx1)
        rby = jnp.minimum(y1, ty1)
        inter = jnp.maximum(rbx - ltx, 0.0) * jnp.maximum(rby - lty, 0.0)
        union = av + ta - inter
        iou = inter / jnp.maximum(union, 1e-9)
        lcx = jnp.minimum(x0, tx0)
        lcy = jnp.minimum(y0, ty0)
        rcx = jnp.maximum(x1, tx1)
        rcy = jnp.maximum(y1, ty1)
        areac = jnp.maximum(rcx - lcx, 0.0) * jnp.maximum(rcy - lcy, 0.0)
        gv = iou - (areac - union) / jnp.maximum(areac, 1e-9)
        pidx = iota + k * 128
        upd = gv > m
        m = jnp.where(upd, gv, m)
        bi = jnp.where(upd, pidx, bi)
        return m, bi

    m0 = jnp.full((8, 128), -3.0e38, jnp.float32)
    bi0 = jnp.zeros((8, 128), jnp.int32)
    m, bi = lax.fori_loop(0, NT, body, (m0, bi0))
    gm = jnp.max(m, axis=1, keepdims=True)
    cand = jnp.where(m == gm, bi, BIG)
    gi = jnp.min(cand, axis=1, keepdims=True)
    omax_ref[0, 0] = jnp.broadcast_to(gm, (8, 128))
    oidx_ref[0, 0] = jnp.broadcast_to(gi, (8, 128))


_tc_call = pl.pallas_call(
    _tc_body,
    grid=(B, TG),
    in_specs=[
        pl.BlockSpec((1, 4, NT, 128), lambda i, g: (i, 0, 0, 0)),
        pl.BlockSpec((1, TG * 8, 4), lambda i, g: (i, 0, 0)),
    ],
    out_specs=[
        pl.BlockSpec((1, 1, 8, 128), lambda i, g: (i, g, 0, 0)),
        pl.BlockSpec((1, 1, 8, 128), lambda i, g: (i, g, 0, 0)),
    ],
    out_shape=[
        jax.ShapeDtypeStruct((B, TG, 8, 128), jnp.float32),
        jax.ShapeDtypeStruct((B, TG, 8, 128), jnp.int32),
    ],
    scratch_shapes=[pltpu.VMEM((NT, 128), jnp.float32) for _ in range(5)],
)


@jax.jit
def _matcher(pred_boxes, target_boxes):
    sc = pl.kernel(
        _sc_body,
        out_type=[
            jax.ShapeDtypeStruct((B * TPAD,), jnp.float32),
            jax.ShapeDtypeStruct((B * TPAD,), jnp.int32),
        ],
        mesh=plsc.VectorSubcoreMesh(core_axis_name="c", subcore_axis_name="s",
                                    num_cores=NCORE, num_subcores=NSUB),
        compiler_params=pltpu.CompilerParams(needs_layout_passes=False),
        scratch_types=[
            pltpu.VMEM((PER_W * 4,), jnp.float32),
            pltpu.VMEM((PER_W,), jnp.float32),
            pltpu.VMEM((PER_W,), jnp.float32),
            pltpu.VMEM((PER_W,), jnp.float32),
            pltpu.VMEM((PER_W,), jnp.float32),
            pltpu.VMEM((PER_W,), jnp.float32),
            pltpu.VMEM((TPAD * 4,), jnp.float32),
            pltpu.VMEM((B_PER_CORE * TPAD,), jnp.float32),
            pltpu.VMEM((B_PER_CORE * TPAD,), jnp.int32),
            pltpu.VMEM((NSUB * TPAD,), jnp.float32),
            pltpu.VMEM((NSUB * TPAD,), jnp.int32),
            pltpu.VMEM((TPAD,), jnp.float32),
            pltpu.VMEM((TPAD,), jnp.int32),
            pltpu.VMEM_SHARED((NSUB * B_PER_CORE * TPAD,), jnp.float32),
            pltpu.VMEM_SHARED((NSUB * B_PER_CORE * TPAD,), jnp.int32),
        ],
    )
    sm, si = sc(pred_boxes.reshape(B * Q * 4), target_boxes.reshape(B * T * 4))
    sm = sm.reshape(B, TPAD)[:, :T]
    si = si.reshape(B, TPAD)[:, :T]

    ptc = pred_boxes[:, QS:, :].transpose(0, 2, 1).reshape(B, 4, NT, 128)
    tpad = jnp.pad(target_boxes, ((0, 0), (0, TG * 8 - T), (0, 0)))
    tm4, ti4 = _tc_call(ptc, tpad)
    tm = tm4[:, :, :, 0].reshape(B, TG * 8)[:, :T]
    ti = ti4[:, :, :, 0].reshape(B, TG * 8)[:, :T] + QS

    use_sc = sm >= tm
    max_iou = jnp.where(use_sc, sm, tm)
    pred_idx = jnp.where(use_sc, si, ti)
    return pred_idx, max_iou


def kernel(pred_boxes, target_boxes):
    pred_idx, max_iou = _matcher(pred_boxes, target_boxes)
    valid = max_iou >= 0.5
    return pred_idx, valid, max_iou


# speedup vs baseline: 4.4971x; 4.4971x over previous
import functools

import jax
import jax.numpy as jnp
from jax import lax
from jax.experimental import pallas as pl
from jax.experimental.pallas import tpu as pltpu
from jax.experimental.pallas import tpu_sc as plsc

B = 8
Q = 20000
T = 100
TPAD = 112
NCORE = 2
NSUB = 16
B_PER_CORE = B // NCORE
G = 4

QS = 6688
QT = Q - QS
NT = QT // 128
TC_U = 4
TG = 13
PER_W = 432
CHUNKS = PER_W // 16
BIG = 1 << 30


def _sc_body(pred_hbm, tgt_hbm, outmax_hbm, outidx_hbm,
             praw, x0a, y0a, x1a, y1a, aa, traw,
             lmax, lidx, mgmax, mgidx, omax, oidx,
             shmax, shidx):
    c = lax.axis_index("c")
    s = lax.axis_index("s")
    iota = lax.iota(jnp.int32, 16)
    lane0 = iota == 0
    zc = jnp.zeros((16,), jnp.int32)
    base = jnp.minimum(s * PER_W, QS - PER_W)

    for bl in range(B_PER_CORE):
        b = c * B_PER_CORE + bl
        pltpu.sync_copy(pred_hbm.at[pl.ds(b * (Q * 4) + base * 4, PER_W * 4)], praw)
        pltpu.sync_copy(tgt_hbm.at[pl.ds(b * (T * 4), T * 4)], traw.at[pl.ds(0, T * 4)])

        def pre(j, _):
            r = j * 64 + iota * 4
            cx = plsc.load_gather(praw, [r])
            cy = plsc.load_gather(praw, [r + 1])
            w = plsc.load_gather(praw, [r + 2])
            h = plsc.load_gather(praw, [r + 3])
            x0 = cx - 0.5 * w
            y0 = cy - 0.5 * h
            x1 = cx + 0.5 * w
            y1 = cy + 0.5 * h
            sl = pl.ds(j * 16, 16)
            x0a[sl] = x0
            y0a[sl] = y0
            x1a[sl] = x1
            y1a[sl] = y1
            aa[sl] = (x1 - x0) * (y1 - y0)
            return 0

        lax.fori_loop(0, CHUNKS, pre, 0)

        def per_g(g, _):
            t0 = g * G
            tco = []
            for i in range(G):
                t4 = zc + (t0 + i) * 4
                tcx = plsc.load_gather(traw, [t4])
                tcy = plsc.load_gather(traw, [t4 + 1])
                tw = plsc.load_gather(traw, [t4 + 2])
                th = plsc.load_gather(traw, [t4 + 3])
                tx0 = tcx - 0.5 * tw
                ty0 = tcy - 0.5 * th
                tx1 = tcx + 0.5 * tw
                ty1 = tcy + 0.5 * th
                ta = (tx1 - tx0) * (ty1 - ty0)
                tco.append((tx0, ty0, tx1, ty1, ta))

            def scan_k(k, carry):
                ms, bis, idxv = carry
                sl = pl.ds(k * 16, 16)
                x0 = x0a[sl]
                y0 = y0a[sl]
                x1 = x1a[sl]
                y1 = y1a[sl]
                av = aa[sl]
                nms, nbis = [], []
                for i in range(G):
                    tx0, ty0, tx1, ty1, ta = tco[i]
                    ltx = jnp.maximum(x0, tx0)
                    lty = jnp.maximum(y0, ty0)
                    rbx = jnp.minimum(x1, tx1)
                    rby = jnp.minimum(y1, ty1)
                    inter = jnp.maximum(rbx - ltx, 0.0) * jnp.maximum(rby - lty, 0.0)
                    union = av + ta - inter
                    iou = inter / jnp.maximum(union, 1e-9)
                    lcx = jnp.minimum(x0, tx0)
                    lcy = jnp.minimum(y0, ty0)
                    rcx = jnp.maximum(x1, tx1)
                    rcy = jnp.maximum(y1, ty1)
                    areac = jnp.maximum(rcx - lcx, 0.0) * jnp.maximum(rcy - lcy, 0.0)
                    gv = iou - (areac - union) / jnp.maximum(areac, 1e-9)
                    upd = gv > ms[i]
                    nms.append(jnp.where(upd, gv, ms[i]))
                    nbis.append(jnp.where(upd, idxv, bis[i]))
                return tuple(nms), tuple(nbis), idxv + 16

            m0 = jnp.full((16,), -3.0e38, jnp.float32)
            bi0 = jnp.zeros((16,), jnp.int32)
            ms, bis, _ = lax.fori_loop(
                0, CHUNKS, scan_k,
                ((m0,) * G, (bi0,) * G, base + iota))
            for i in range(G):
                gm = jnp.max(ms[i])
                cand = jnp.where(ms[i] == jnp.full((16,), gm),
                                 bis[i], jnp.full((16,), BIG, jnp.int32))
                gi = jnp.min(cand)
                posv = zc + (bl * TPAD + t0 + i)
                plsc.store_scatter(lmax, [posv], jnp.full((16,), gm), mask=lane0)
                plsc.store_scatter(lidx, [posv], jnp.full((16,), gi, jnp.int32),
                                   mask=lane0)
            return 0

        lax.fori_loop(0, T // G, per_g, 0)

    pltpu.sync_copy(lmax, shmax.at[pl.ds(s * (B_PER_CORE * TPAD), B_PER_CORE * TPAD)])
    pltpu.sync_copy(lidx, shidx.at[pl.ds(s * (B_PER_CORE * TPAD), B_PER_CORE * TPAD)])
    plsc.subcore_barrier()

    @pl.when(s < B_PER_CORE)
    def _merge():
        for w in range(NSUB):
            pltpu.sync_copy(shmax.at[pl.ds(w * (B_PER_CORE * TPAD) + s * TPAD, TPAD)],
                            mgmax.at[pl.ds(w * TPAD, TPAD)])
            pltpu.sync_copy(shidx.at[pl.ds(w * (B_PER_CORE * TPAD) + s * TPAD, TPAD)],
                            mgidx.at[pl.ds(w * TPAD, TPAD)])

        def mg(cc, _):
            sl = pl.ds(cc * 16, 16)
            del _
            acc = mgmax[sl]
            acci = mgidx[sl]
            for w in range(1, NSUB):
                wsl = pl.ds(w * TPAD + cc * 16, 16)
                v = mgmax[wsl]
                vi = mgidx[wsl]
                upd = (v > acc) | ((v == acc) & (vi < acci))
                acc = jnp.where(upd, v, acc)
                acci = jnp.where(upd, vi, acci)
            omax[sl] = acc
            oidx[sl] = acci
            return 0

        lax.fori_loop(0, TPAD // 16, mg, 0)
        gb = c * B_PER_CORE + s
        pltpu.sync_copy(omax, outmax_hbm.at[pl.ds(gb * TPAD, TPAD)])
        pltpu.sync_copy(oidx, outidx_hbm.at[pl.ds(gb * TPAD, TPAD)])


def _tc_body(pref, tref, omax_ref, oidx_ref, x0s, y0s, x1s, y1s, aas):
    g = pl.program_id(1)

    @pl.when(g == 0)
    def _pre():
        cx = pref[0, 0]
        cy = pref[0, 1]
        w = pref[0, 2]
        h = pref[0, 3]
        x0 = cx - 0.5 * w
        y0 = cy - 0.5 * h
        x1 = cx + 0.5 * w
        y1 = cy + 0.5 * h
        x0s[...] = x0
        y0s[...] = y0
        x1s[...] = x1
        y1s[...] = y1
        aas[...] = (x1 - x0) * (y1 - y0)

    tcx = tref[0, pl.ds(g * 8, 8), 0:1]
    tcy = tref[0, pl.ds(g * 8, 8), 1:2]
    tw = tref[0, pl.ds(g * 8, 8), 2:3]
    th = tref[0, pl.ds(g * 8, 8), 3:4]
    tx0 = jnp.broadcast_to(tcx - 0.5 * tw, (8, 128))
    ty0 = jnp.broadcast_to(tcy - 0.5 * th, (8, 128))
    tx1 = jnp.broadcast_to(tcx + 0.5 * tw, (8, 128))
    ty1 = jnp.broadcast_to(tcy + 0.5 * th, (8, 128))
    ta = (tx1 - tx0) * (ty1 - ty0)
    iota = lax.broadcasted_iota(jnp.int32, (8, 128), 1)

    def body(k, carry):
        ms, bis = carry
        nms, nbis = [], []
        for u in range(TC_U):
            kk = k * TC_U + u
            x0 = x0s[pl.ds(kk, 1), :]
            y0 = y0s[pl.ds(kk, 1), :]
            x1 = x1s[pl.ds(kk, 1), :]
            y1 = y1s[pl.ds(kk, 1), :]
            av = aas[pl.ds(kk, 1), :]
            ltx = jnp.maximum(x0, tx0)
            lty = jnp.maximum(y0, ty0)
            rbx = jnp.minimum(x1, tx1)
            rby = jnp.minimum(y1, ty1)
            inter = jnp.maximum(rbx - ltx, 0.0) * jnp.maximum(rby - lty, 0.0)
            union = av + ta - inter
            iou = inter / jnp.maximum(union, 1e-9)
            lcx = jnp.minimum(x0, tx0)
            lcy = jnp.minimum(y0, ty0)
            rcx = jnp.maximum(x1, tx1)
            rcy = jnp.maximum(y1, ty1)
            areac = jnp.maximum(rcx - lcx, 0.0) * jnp.maximum(rcy - lcy, 0.0)
            gv = iou - (areac - union) / jnp.maximum(areac, 1e-9)
            pidx = iota + kk * 128
            upd = gv > ms[u]
            nms.append(jnp.where(upd, gv, ms[u]))
            nbis.append(jnp.where(upd, pidx, bis[u]))
        return tuple(nms), tuple(nbis)

    m0 = jnp.full((8, 128), -3.0e38, jnp.float32)
    bi0 = jnp.zeros((8, 128), jnp.int32)
    ms, bis = lax.fori_loop(0, NT // TC_U, body,
                            ((m0,) * TC_U, (bi0,) * TC_U))
    m, bi = ms[0], bis[0]
    for u in range(1, TC_U):
        upd = (ms[u] > m) | ((ms[u] == m) & (bis[u] < bi))
        m = jnp.where(upd, ms[u], m)
        bi = jnp.where(upd, bis[u], bi)
    gm = jnp.max(m, axis=1, keepdims=True)
    cand = jnp.where(m == gm, bi, BIG)
    gi = jnp.min(cand, axis=1, keepdims=True)
    omax_ref[0, 0] = jnp.broadcast_to(gm, (8, 128))
    oidx_ref[0, 0] = jnp.broadcast_to(gi, (8, 128))


_tc_call = pl.pallas_call(
    _tc_body,
    grid=(B, TG),
    in_specs=[
        pl.BlockSpec((1, 4, NT, 128), lambda i, g: (i, 0, 0, 0)),
        pl.BlockSpec((1, TG * 8, 4), lambda i, g: (i, 0, 0)),
    ],
    out_specs=[
        pl.BlockSpec((1, 1, 8, 128), lambda i, g: (i, g, 0, 0)),
        pl.BlockSpec((1, 1, 8, 128), lambda i, g: (i, g, 0, 0)),
    ],
    out_shape=[
        jax.ShapeDtypeStruct((B, TG, 8, 128), jnp.float32),
        jax.ShapeDtypeStruct((B, TG, 8, 128), jnp.int32),
    ],
    scratch_shapes=[pltpu.VMEM((NT, 128), jnp.float32) for _ in range(5)],
)


@jax.jit
def _matcher(pred_boxes, target_boxes):
    sc = pl.kernel(
        _sc_body,
        out_type=[
            jax.ShapeDtypeStruct((B * TPAD,), jnp.float32),
            jax.ShapeDtypeStruct((B * TPAD,), jnp.int32),
        ],
        mesh=plsc.VectorSubcoreMesh(core_axis_name="c", subcore_axis_name="s",
                                    num_cores=NCORE, num_subcores=NSUB),
        compiler_params=pltpu.CompilerParams(needs_layout_passes=False),
        scratch_types=[
            pltpu.VMEM((PER_W * 4,), jnp.float32),
            pltpu.VMEM((PER_W,), jnp.float32),
            pltpu.VMEM((PER_W,), jnp.float32),
            pltpu.VMEM((PER_W,), jnp.float32),
            pltpu.VMEM((PER_W,), jnp.float32),
            pltpu.VMEM((PER_W,), jnp.float32),
            pltpu.VMEM((TPAD * 4,), jnp.float32),
            pltpu.VMEM((B_PER_CORE * TPAD,), jnp.float32),
            pltpu.VMEM((B_PER_CORE * TPAD,), jnp.int32),
            pltpu.VMEM((NSUB * TPAD,), jnp.float32),
            pltpu.VMEM((NSUB * TPAD,), jnp.int32),
            pltpu.VMEM((TPAD,), jnp.float32),
            pltpu.VMEM((TPAD,), jnp.int32),
            pltpu.VMEM_SHARED((NSUB * B_PER_CORE * TPAD,), jnp.float32),
            pltpu.VMEM_SHARED((NSUB * B_PER_CORE * TPAD,), jnp.int32),
        ],
    )
    sm, si = sc(pred_boxes.reshape(B * Q * 4), target_boxes.reshape(B * T * 4))
    sm = sm.reshape(B, TPAD)[:, :T]
    si = si.reshape(B, TPAD)[:, :T]

    ptc = pred_boxes[:, QS:, :].transpose(0, 2, 1).reshape(B, 4, NT, 128)
    tpad = jnp.pad(target_boxes, ((0, 0), (0, TG * 8 - T), (0, 0)))
    tm4, ti4 = _tc_call(ptc, tpad)
    tm = tm4[:, :, :, 0].reshape(B, TG * 8)[:, :T]
    ti = ti4[:, :, :, 0].reshape(B, TG * 8)[:, :T] + QS

    use_sc = sm >= tm
    max_iou = jnp.where(use_sc, sm, tm)
    pred_idx = jnp.where(use_sc, si, ti)
    return pred_idx, max_iou


def kernel(pred_boxes, target_boxes):
    pred_idx, max_iou = _matcher(pred_boxes, target_boxes)
    valid = max_iou >= 0.5
    return pred_idx, valid, max_iou


# speedup vs baseline: 5.9762x; 1.3289x over previous
import functools

import jax
import jax.numpy as jnp
from jax import lax
from jax.experimental import pallas as pl
from jax.experimental.pallas import tpu as pltpu
from jax.experimental.pallas import tpu_sc as plsc

B = 8
Q = 20000
T = 100
TPAD = 112
NCORE = 2
NSUB = 16
B_PER_CORE = B // NCORE
G = 4

QS = 8192
PER_W = 512
CHUNKS = PER_W // 16
NT = 96
QTC0 = Q - NT * 128
TC_U = 4
TG = 13
BIG = 1 << 30


def _sc_body(pred_hbm, tgt_hbm, outmax_hbm, outidx_hbm,
             praw8, x0a, y0a, x1a, y1a, aa, traw,
             lmax, lidx, mgmax, mgidx, omax, oidx,
             shmax, shidx):
    c = lax.axis_index("c")
    s = lax.axis_index("s")
    iota = lax.iota(jnp.int32, 16)
    lane0 = iota == 0
    zc = jnp.zeros((16,), jnp.int32)
    base = s * PER_W

    pltpu.sync_copy(pred_hbm.at[:, pl.ds(base * 4, PER_W * 4)], praw8)

    for bl in range(B_PER_CORE):
        b = c * B_PER_CORE + bl
        pltpu.sync_copy(tgt_hbm.at[pl.ds(b * (T * 4), T * 4)],
                        traw.at[pl.ds(0, T * 4)])
        brow = zc + b

        def pre(j, _):
            r = j * 64 + iota * 4
            cx = plsc.load_gather(praw8, [brow, r])
            cy = plsc.load_gather(praw8, [brow, r + 1])
            w = plsc.load_gather(praw8, [brow, r + 2])
            h = plsc.load_gather(praw8, [brow, r + 3])
            x0 = cx - 0.5 * w
            y0 = cy - 0.5 * h
            x1 = cx + 0.5 * w
            y1 = cy + 0.5 * h
            sl = pl.ds(j * 16, 16)
            x0a[sl] = x0
            y0a[sl] = y0
            x1a[sl] = x1
            y1a[sl] = y1
            aa[sl] = (x1 - x0) * (y1 - y0)
            return 0

        lax.fori_loop(0, CHUNKS, pre, 0)

        def per_g(g, _):
            t0 = g * G
            tco = []
            for i in range(G):
                t4 = zc + (t0 + i) * 4
                tcx = plsc.load_gather(traw, [t4])
                tcy = plsc.load_gather(traw, [t4 + 1])
                tw = plsc.load_gather(traw, [t4 + 2])
                th = plsc.load_gather(traw, [t4 + 3])
                tx0 = tcx - 0.5 * tw
                ty0 = tcy - 0.5 * th
                tx1 = tcx + 0.5 * tw
                ty1 = tcy + 0.5 * th
                ta = (tx1 - tx0) * (ty1 - ty0)
                tco.append((tx0, ty0, tx1, ty1, ta))

            def scan_k(k, carry):
                ms, bis, idxv = carry
                sl = pl.ds(k * 16, 16)
                x0 = x0a[sl]
                y0 = y0a[sl]
                x1 = x1a[sl]
                y1 = y1a[sl]
                av = aa[sl]
                nms, nbis = [], []
                for i in range(G):
                    tx0, ty0, tx1, ty1, ta = tco[i]
                    ltx = jnp.maximum(x0, tx0)
                    lty = jnp.maximum(y0, ty0)
                    rbx = jnp.minimum(x1, tx1)
                    rby = jnp.minimum(y1, ty1)
                    inter = jnp.maximum(rbx - ltx, 0.0) * jnp.maximum(rby - lty, 0.0)
                    union = av + ta - inter
                    iou = inter / jnp.maximum(union, 1e-9)
                    lcx = jnp.minimum(x0, tx0)
                    lcy = jnp.minimum(y0, ty0)
                    rcx = jnp.maximum(x1, tx1)
                    rcy = jnp.maximum(y1, ty1)
                    areac = jnp.maximum(rcx - lcx, 0.0) * jnp.maximum(rcy - lcy, 0.0)
                    gv = iou - (areac - union) / jnp.maximum(areac, 1e-9)
                    upd = gv > ms[i]
                    nms.append(jnp.where(upd, gv, ms[i]))
                    nbis.append(jnp.where(upd, idxv, bis[i]))
                return tuple(nms), tuple(nbis), idxv + 16

            m0 = jnp.full((16,), -3.0e38, jnp.float32)
            bi0 = jnp.zeros((16,), jnp.int32)
            ms, bis, _ = lax.fori_loop(
                0, CHUNKS, scan_k,
                ((m0,) * G, (bi0,) * G, base + iota))
            for i in range(G):
                gm = jnp.max(ms[i])
                cand = jnp.where(ms[i] == jnp.full((16,), gm),
                                 bis[i], jnp.full((16,), BIG, jnp.int32))
                gi = jnp.min(cand)
                posv = zc + (bl * TPAD + t0 + i)
                plsc.store_scatter(lmax, [posv], jnp.full((16,), gm), mask=lane0)
                plsc.store_scatter(lidx, [posv], jnp.full((16,), gi, jnp.int32),
                                   mask=lane0)
            return 0

        lax.fori_loop(0, T // G, per_g, 0)

    pltpu.sync_copy(lmax, shmax.at[pl.ds(s * (B_PER_CORE * TPAD), B_PER_CORE * TPAD)])
    pltpu.sync_copy(lidx, shidx.at[pl.ds(s * (B_PER_CORE * TPAD), B_PER_CORE * TPAD)])
    plsc.subcore_barrier()

    @pl.when(s < B_PER_CORE)
    def _merge():
        for w in range(NSUB):
            pltpu.sync_copy(shmax.at[pl.ds(w * (B_PER_CORE * TPAD) + s * TPAD, TPAD)],
                            mgmax.at[pl.ds(w * TPAD, TPAD)])
            pltpu.sync_copy(shidx.at[pl.ds(w * (B_PER_CORE * TPAD) + s * TPAD, TPAD)],
                            mgidx.at[pl.ds(w * TPAD, TPAD)])

        def mg(cc, _):
            sl = pl.ds(cc * 16, 16)
            del _
            acc = mgmax[sl]
            acci = mgidx[sl]
            for w in range(1, NSUB):
                wsl = pl.ds(w * TPAD + cc * 16, 16)
                v = mgmax[wsl]
                vi = mgidx[wsl]
                upd = (v > acc) | ((v == acc) & (vi < acci))
                acc = jnp.where(upd, v, acc)
                acci = jnp.where(upd, vi, acci)
            omax[sl] = acc
            oidx[sl] = acci
            return 0

        lax.fori_loop(0, TPAD // 16, mg, 0)
        gb = c * B_PER_CORE + s
        pltpu.sync_copy(omax, outmax_hbm.at[pl.ds(gb * TPAD, TPAD)])
        pltpu.sync_copy(oidx, outidx_hbm.at[pl.ds(gb * TPAD, TPAD)])


def _tc_body(cxref, cyref, wref, href, tref, omax_ref, oidx_ref,
             x0s, y0s, x1s, y1s, aas):
    g = pl.program_id(1)

    @pl.when(g == 0)
    def _pre():
        cx = cxref[0]
        cy = cyref[0]
        w = wref[0]
        h = href[0]
        x0 = cx - 0.5 * w
        y0 = cy - 0.5 * h
        x1 = cx + 0.5 * w
        y1 = cy + 0.5 * h
        x0s[...] = x0
        y0s[...] = y0
        x1s[...] = x1
        y1s[...] = y1
        aas[...] = (x1 - x0) * (y1 - y0)

    tcx = tref[0, pl.ds(g * 8, 8), 0:1]
    tcy = tref[0, pl.ds(g * 8, 8), 1:2]
    tw = tref[0, pl.ds(g * 8, 8), 2:3]
    th = tref[0, pl.ds(g * 8, 8), 3:4]
    tx0 = jnp.broadcast_to(tcx - 0.5 * tw, (8, 128))
    ty0 = jnp.broadcast_to(tcy - 0.5 * th, (8, 128))
    tx1 = jnp.broadcast_to(tcx + 0.5 * tw, (8, 128))
    ty1 = jnp.broadcast_to(tcy + 0.5 * th, (8, 128))
    ta = (tx1 - tx0) * (ty1 - ty0)
    iota = lax.broadcasted_iota(jnp.int32, (8, 128), 1)

    def body(k, carry):
        ms, bis = carry
        nms, nbis = [], []
        for u in range(TC_U):
            kk = k * TC_U + u
            x0 = x0s[pl.ds(kk, 1), :]
            y0 = y0s[pl.ds(kk, 1), :]
            x1 = x1s[pl.ds(kk, 1), :]
            y1 = y1s[pl.ds(kk, 1), :]
            av = aas[pl.ds(kk, 1), :]
            ltx = jnp.maximum(x0, tx0)
            lty = jnp.maximum(y0, ty0)
            rbx = jnp.minimum(x1, tx1)
            rby = jnp.minimum(y1, ty1)
            inter = jnp.maximum(rbx - ltx, 0.0) * jnp.maximum(rby - lty, 0.0)
            union = av + ta - inter
            iou = inter / jnp.maximum(union, 1e-9)
            lcx = jnp.minimum(x0, tx0)
            lcy = jnp.minimum(y0, ty0)
            rcx = jnp.maximum(x1, tx1)
            rcy = jnp.maximum(y1, ty1)
            areac = jnp.maximum(rcx - lcx, 0.0) * jnp.maximum(rcy - lcy, 0.0)
            gv = iou - (areac - union) / jnp.maximum(areac, 1e-9)
            pidx = iota + kk * 128
            upd = gv > ms[u]
            nms.append(jnp.where(upd, gv, ms[u]))
            nbis.append(jnp.where(upd, pidx, bis[u]))
        return tuple(nms), tuple(nbis)

    m0 = jnp.full((8, 128), -3.0e38, jnp.float32)
    bi0 = jnp.zeros((8, 128), jnp.int32)
    ms, bis = lax.fori_loop(0, NT // TC_U, body,
                            ((m0,) * TC_U, (bi0,) * TC_U))
    m, bi = ms[0], bis[0]
    for u in range(1, TC_U):
        upd = (ms[u] > m) | ((ms[u] == m) & (bis[u] < bi))
        m = jnp.where(upd, ms[u], m)
        bi = jnp.where(upd, bis[u], bi)
    gm = jnp.max(m, axis=1, keepdims=True)
    cand = jnp.where(m == gm, bi, BIG)
    gi = jnp.min(cand, axis=1, keepdims=True)
    omax_ref[0, 0] = jnp.broadcast_to(gm, (8, 128))
    oidx_ref[0, 0] = jnp.broadcast_to(gi, (8, 128))


_tc_call = pl.pallas_call(
    _tc_body,
    grid=(B, TG),
    in_specs=[
        pl.BlockSpec((1, NT, 128), lambda i, g: (i, 0, 0)),
        pl.BlockSpec((1, NT, 128), lambda i, g: (i, 0, 0)),
        pl.BlockSpec((1, NT, 128), lambda i, g: (i, 0, 0)),
        pl.BlockSpec((1, NT, 128), lambda i, g: (i, 0, 0)),
        pl.BlockSpec((1, TG * 8, 4), lambda i, g: (i, 0, 0)),
    ],
    out_specs=[
        pl.BlockSpec((1, 1, 8, 128), lambda i, g: (i, g, 0, 0)),
        pl.BlockSpec((1, 1, 8, 128), lambda i, g: (i, g, 0, 0)),
    ],
    out_shape=[
        jax.ShapeDtypeStruct((B, TG, 8, 128), jnp.float32),
        jax.ShapeDtypeStruct((B, TG, 8, 128), jnp.int32),
    ],
    scratch_shapes=[pltpu.VMEM((NT, 128), jnp.float32) for _ in range(5)],
)


@jax.jit
def _matcher(pred_boxes, target_boxes):
    sc = pl.kernel(
        _sc_body,
        out_type=[
            jax.ShapeDtypeStruct((B * TPAD,), jnp.float32),
            jax.ShapeDtypeStruct((B * TPAD,), jnp.int32),
        ],
        mesh=plsc.VectorSubcoreMesh(core_axis_name="c", subcore_axis_name="s",
                                    num_cores=NCORE, num_subcores=NSUB),
        compiler_params=pltpu.CompilerParams(needs_layout_passes=False),
        scratch_types=[
            pltpu.VMEM((B, PER_W * 4), jnp.float32),
            pltpu.VMEM((PER_W,), jnp.float32),
            pltpu.VMEM((PER_W,), jnp.float32),
            pltpu.VMEM((PER_W,), jnp.float32),
            pltpu.VMEM((PER_W,), jnp.float32),
            pltpu.VMEM((PER_W,), jnp.float32),
            pltpu.VMEM((TPAD * 4,), jnp.float32),
            pltpu.VMEM((B_PER_CORE * TPAD,), jnp.float32),
            pltpu.VMEM((B_PER_CORE * TPAD,), jnp.int32),
            pltpu.VMEM((NSUB * TPAD,), jnp.float32),
            pltpu.VMEM((NSUB * TPAD,), jnp.int32),
            pltpu.VMEM((TPAD,), jnp.float32),
            pltpu.VMEM((TPAD,), jnp.int32),
            pltpu.VMEM_SHARED((NSUB * B_PER_CORE * TPAD,), jnp.float32),
            pltpu.VMEM_SHARED((NSUB * B_PER_CORE * TPAD,), jnp.int32),
        ],
    )
    sm, si = sc(pred_boxes.reshape(B, Q * 4), target_boxes.reshape(B * T * 4))
    sm = sm.reshape(B, TPAD)[:, :T]
    si = si.reshape(B, TPAD)[:, :T]

    ptc = pred_boxes[:, QTC0:, :]
    cx4 = ptc[:, :, 0].reshape(B, NT, 128)
    cy4 = ptc[:, :, 1].reshape(B, NT, 128)
    w4 = ptc[:, :, 2].reshape(B, NT, 128)
    h4 = ptc[:, :, 3].reshape(B, NT, 128)
    tpad = jnp.pad(target_boxes, ((0, 0), (0, TG * 8 - T), (0, 0)))
    tm4, ti4 = _tc_call(cx4, cy4, w4, h4, tpad)
    tm = tm4[:, :, :, 0].reshape(B, TG * 8)[:, :T]
    ti = ti4[:, :, :, 0].reshape(B, TG * 8)[:, :T] + QTC0

    use_sc = sm >= tm
    max_iou = jnp.where(use_sc, sm, tm)
    pred_idx = jnp.where(use_sc, si, ti)
    return pred_idx, max_iou


def kernel(pred_boxes, target_boxes):
    pred_idx, max_iou = _matcher(pred_boxes, target_boxes)
    valid = max_iou >= 0.5
    return pred_idx, valid, max_iou


# speedup vs baseline: 7.2451x; 1.2123x over previous
import functools

import jax
import jax.numpy as jnp
from jax import lax
from jax.experimental import pallas as pl
from jax.experimental.pallas import tpu as pltpu
from jax.experimental.pallas import tpu_sc as plsc

B = 8
Q = 20000
T = 100
TPAD = 112
NCORE = 2
NSUB = 16
B_PER_CORE = B // NCORE
G = 4

QS = 8192
PER_W = 512
CHUNKS = PER_W // 16
NT = 96
QTC0 = Q - NT * 128
TC_U = 4
TG = 13
BIG = 1 << 30


def _sc_body(pred_hbm, tgt_hbm, outmax_hbm, outidx_hbm,
             praw8, x0a, y0a, x1a, y1a, aa, traw,
             lmax, lidx, mgmax, mgidx, omax, oidx,
             shmax, shidx):
    c = lax.axis_index("c")
    s = lax.axis_index("s")
    iota = lax.iota(jnp.int32, 16)
    lane0 = iota == 0
    zc = jnp.zeros((16,), jnp.int32)
    base = s * PER_W

    pltpu.sync_copy(pred_hbm.at[:, pl.ds(base * 4, PER_W * 4)], praw8)

    for bl in range(B_PER_CORE):
        b = c * B_PER_CORE + bl
        pltpu.sync_copy(tgt_hbm.at[b], traw)
        brow = zc + b

        def pre(j, _):
            r = j * 64 + iota * 4
            cx = plsc.load_gather(praw8, [brow, r])
            cy = plsc.load_gather(praw8, [brow, r + 1])
            w = plsc.load_gather(praw8, [brow, r + 2])
            h = plsc.load_gather(praw8, [brow, r + 3])
            x0 = cx - 0.5 * w
            y0 = cy - 0.5 * h
            x1 = cx + 0.5 * w
            y1 = cy + 0.5 * h
            sl = pl.ds(j * 16, 16)
            x0a[sl] = x0
            y0a[sl] = y0
            x1a[sl] = x1
            y1a[sl] = y1
            aa[sl] = (x1 - x0) * (y1 - y0)
            return 0

        lax.fori_loop(0, CHUNKS, pre, 0)

        def per_g(g, _):
            t0 = g * G
            tco = []
            for i in range(G):
                t4 = zc + (t0 + i) * 4
                tcx = plsc.load_gather(traw, [t4])
                tcy = plsc.load_gather(traw, [t4 + 1])
                tw = plsc.load_gather(traw, [t4 + 2])
                th = plsc.load_gather(traw, [t4 + 3])
                tx0 = tcx - 0.5 * tw
                ty0 = tcy - 0.5 * th
                tx1 = tcx + 0.5 * tw
                ty1 = tcy + 0.5 * th
                ta = (tx1 - tx0) * (ty1 - ty0)
                tco.append((tx0, ty0, tx1, ty1, ta))

            def scan_k(k, carry):
                ms, bis, idxv = carry
                sl = pl.ds(k * 16, 16)
                x0 = x0a[sl]
                y0 = y0a[sl]
                x1 = x1a[sl]
                y1 = y1a[sl]
                av = aa[sl]
                nms, nbis = [], []
                for i in range(G):
                    tx0, ty0, tx1, ty1, ta = tco[i]
                    ltx = jnp.maximum(x0, tx0)
                    lty = jnp.maximum(y0, ty0)
                    rbx = jnp.minimum(x1, tx1)
                    rby = jnp.minimum(y1, ty1)
                    inter = jnp.maximum(rbx - ltx, 0.0) * jnp.maximum(rby - lty, 0.0)
                    union = av + ta - inter
                    iou = inter / jnp.maximum(union, 1e-9)
                    lcx = jnp.minimum(x0, tx0)
                    lcy = jnp.minimum(y0, ty0)
                    rcx = jnp.maximum(x1, tx1)
                    rcy = jnp.maximum(y1, ty1)
                    areac = (rcx - lcx) * (rcy - lcy)
                    gv = iou - (areac - union) / jnp.maximum(areac, 1e-9)
                    upd = gv > ms[i]
                    nms.append(jnp.where(upd, gv, ms[i]))
                    nbis.append(jnp.where(upd, idxv, bis[i]))
                return tuple(nms), tuple(nbis), idxv + 16

            m0 = jnp.full((16,), -3.0e38, jnp.float32)
            bi0 = jnp.zeros((16,), jnp.int32)
            ms, bis, _ = lax.fori_loop(
                0, CHUNKS, scan_k,
                ((m0,) * G, (bi0,) * G, base + iota))
            for i in range(G):
                gm = jnp.max(ms[i])
                cand = jnp.where(ms[i] == jnp.full((16,), gm),
                                 bis[i], jnp.full((16,), BIG, jnp.int32))
                gi = jnp.min(cand)
                posv = zc + (bl * TPAD + t0 + i)
                plsc.store_scatter(lmax, [posv], jnp.full((16,), gm), mask=lane0)
                plsc.store_scatter(lidx, [posv], jnp.full((16,), gi, jnp.int32),
                                   mask=lane0)
            return 0

        lax.fori_loop(0, T // G, per_g, 0)

    pltpu.sync_copy(lmax, shmax.at[pl.ds(s * (B_PER_CORE * TPAD), B_PER_CORE * TPAD)])
    pltpu.sync_copy(lidx, shidx.at[pl.ds(s * (B_PER_CORE * TPAD), B_PER_CORE * TPAD)])
    plsc.subcore_barrier()

    @pl.when(s < B_PER_CORE)
    def _merge():
        for w in range(NSUB):
            pltpu.sync_copy(shmax.at[pl.ds(w * (B_PER_CORE * TPAD) + s * TPAD, TPAD)],
                            mgmax.at[pl.ds(w * TPAD, TPAD)])
            pltpu.sync_copy(shidx.at[pl.ds(w * (B_PER_CORE * TPAD) + s * TPAD, TPAD)],
                            mgidx.at[pl.ds(w * TPAD, TPAD)])

        def mg(cc, _):
            sl = pl.ds(cc * 16, 16)
            del _
            acc = mgmax[sl]
            acci = mgidx[sl]
            for w in range(1, NSUB):
                wsl = pl.ds(w * TPAD + cc * 16, 16)
                v = mgmax[wsl]
                vi = mgidx[wsl]
                upd = (v > acc) | ((v == acc) & (vi < acci))
                acc = jnp.where(upd, v, acc)
                acci = jnp.where(upd, vi, acci)
            omax[sl] = acc
            oidx[sl] = acci
            return 0

        lax.fori_loop(0, TPAD // 16, mg, 0)
        gb = c * B_PER_CORE + s
        pltpu.sync_copy(omax, outmax_hbm.at[pl.ds(gb * TPAD, TPAD)])
        pltpu.sync_copy(oidx, outidx_hbm.at[pl.ds(gb * TPAD, TPAD)])


def _tc_body(cxref, cyref, wref, href, tref, omax_ref, oidx_ref,
             x0s, y0s, x1s, y1s, aas):
    g = pl.program_id(1)

    @pl.when(g == 0)
    def _pre():
        cx = cxref[0]
        cy = cyref[0]
        w = wref[0]
        h = href[0]
        x0 = cx - 0.5 * w
        y0 = cy - 0.5 * h
        x1 = cx + 0.5 * w
        y1 = cy + 0.5 * h
        x0s[...] = x0
        y0s[...] = y0
        x1s[...] = x1
        y1s[...] = y1
        aas[...] = (x1 - x0) * (y1 - y0)

    tcx = tref[0, pl.ds(g * 8, 8), 0:1]
    tcy = tref[0, pl.ds(g * 8, 8), 1:2]
    tw = tref[0, pl.ds(g * 8, 8), 2:3]
    th = tref[0, pl.ds(g * 8, 8), 3:4]
    tx0 = jnp.broadcast_to(tcx - 0.5 * tw, (8, 128))
    ty0 = jnp.broadcast_to(tcy - 0.5 * th, (8, 128))
    tx1 = jnp.broadcast_to(tcx + 0.5 * tw, (8, 128))
    ty1 = jnp.broadcast_to(tcy + 0.5 * th, (8, 128))
    ta = (tx1 - tx0) * (ty1 - ty0)
    iota = lax.broadcasted_iota(jnp.int32, (8, 128), 1)

    def body(k, carry):
        ms, bis = carry
        nms, nbis = [], []
        for u in range(TC_U):
            kk = k * TC_U + u
            x0 = x0s[pl.ds(kk, 1), :]
            y0 = y0s[pl.ds(kk, 1), :]
            x1 = x1s[pl.ds(kk, 1), :]
            y1 = y1s[pl.ds(kk, 1), :]
            av = aas[pl.ds(kk, 1), :]
            ltx = jnp.maximum(x0, tx0)
            lty = jnp.maximum(y0, ty0)
            rbx = jnp.minimum(x1, tx1)
            rby = jnp.minimum(y1, ty1)
            inter = jnp.maximum(rbx - ltx, 0.0) * jnp.maximum(rby - lty, 0.0)
            union = av + ta - inter
            iou = inter / jnp.maximum(union, 1e-9)
            lcx = jnp.minimum(x0, tx0)
            lcy = jnp.minimum(y0, ty0)
            rcx = jnp.maximum(x1, tx1)
            rcy = jnp.maximum(y1, ty1)
            areac = (rcx - lcx) * (rcy - lcy)
            gv = iou - (areac - union) / jnp.maximum(areac, 1e-9)
            pidx = iota + kk * 128
            upd = gv > ms[u]
            nms.append(jnp.where(upd, gv, ms[u]))
            nbis.append(jnp.where(upd, pidx, bis[u]))
        return tuple(nms), tuple(nbis)

    m0 = jnp.full((8, 128), -3.0e38, jnp.float32)
    bi0 = jnp.zeros((8, 128), jnp.int32)
    ms, bis = lax.fori_loop(0, NT // TC_U, body,
                            ((m0,) * TC_U, (bi0,) * TC_U))
    m, bi = ms[0], bis[0]
    for u in range(1, TC_U):
        upd = (ms[u] > m) | ((ms[u] == m) & (bis[u] < bi))
        m = jnp.where(upd, ms[u], m)
        bi = jnp.where(upd, bis[u], bi)
    gm = jnp.max(m, axis=1, keepdims=True)
    cand = jnp.where(m == gm, bi, BIG)
    gi = jnp.min(cand, axis=1, keepdims=True)
    omax_ref[0, 0] = jnp.broadcast_to(gm, (8, 128))
    oidx_ref[0, 0] = jnp.broadcast_to(gi, (8, 128))


_tc_call = pl.pallas_call(
    _tc_body,
    grid=(B, TG),
    in_specs=[
        pl.BlockSpec((1, NT, 128), lambda i, g: (i, 0, 0)),
        pl.BlockSpec((1, NT, 128), lambda i, g: (i, 0, 0)),
        pl.BlockSpec((1, NT, 128), lambda i, g: (i, 0, 0)),
        pl.BlockSpec((1, NT, 128), lambda i, g: (i, 0, 0)),
        pl.BlockSpec((1, TG * 8, 4), lambda i, g: (i, 0, 0)),
    ],
    out_specs=[
        pl.BlockSpec((1, 1, 8, 128), lambda i, g: (i, g, 0, 0)),
        pl.BlockSpec((1, 1, 8, 128), lambda i, g: (i, g, 0, 0)),
    ],
    out_shape=[
        jax.ShapeDtypeStruct((B, TG, 8, 128), jnp.float32),
        jax.ShapeDtypeStruct((B, TG, 8, 128), jnp.int32),
    ],
    scratch_shapes=[pltpu.VMEM((NT, 128), jnp.float32) for _ in range(5)],
)


@jax.jit
def _matcher(pred_boxes, target_boxes):
    sc = pl.kernel(
        _sc_body,
        out_type=[
            jax.ShapeDtypeStruct((B * TPAD,), jnp.float32),
            jax.ShapeDtypeStruct((B * TPAD,), jnp.int32),
        ],
        mesh=plsc.VectorSubcoreMesh(core_axis_name="c", subcore_axis_name="s",
                                    num_cores=NCORE, num_subcores=NSUB),
        compiler_params=pltpu.CompilerParams(needs_layout_passes=False),
        scratch_types=[
            pltpu.VMEM((B, PER_W * 4), jnp.float32),
            pltpu.VMEM((PER_W,), jnp.float32),
            pltpu.VMEM((PER_W,), jnp.float32),
            pltpu.VMEM((PER_W,), jnp.float32),
            pltpu.VMEM((PER_W,), jnp.float32),
            pltpu.VMEM((PER_W,), jnp.float32),
            pltpu.VMEM((T * 4,), jnp.float32),
            pltpu.VMEM((B_PER_CORE * TPAD,), jnp.float32),
            pltpu.VMEM((B_PER_CORE * TPAD,), jnp.int32),
            pltpu.VMEM((NSUB * TPAD,), jnp.float32),
            pltpu.VMEM((NSUB * TPAD,), jnp.int32),
            pltpu.VMEM((TPAD,), jnp.float32),
            pltpu.VMEM((TPAD,), jnp.int32),
            pltpu.VMEM_SHARED((NSUB * B_PER_CORE * TPAD,), jnp.float32),
            pltpu.VMEM_SHARED((NSUB * B_PER_CORE * TPAD,), jnp.int32),
        ],
    )
    sm, si = sc(pred_boxes[:, :QS, :].reshape(B, QS * 4),
                target_boxes.reshape(B, T * 4))
    sm = sm.reshape(B, TPAD)[:, :T]
    si = si.reshape(B, TPAD)[:, :T]

    ptc = pred_boxes[:, QTC0:, :]
    cx4 = ptc[:, :, 0].reshape(B, NT, 128)
    cy4 = ptc[:, :, 1].reshape(B, NT, 128)
    w4 = ptc[:, :, 2].reshape(B, NT, 128)
    h4 = ptc[:, :, 3].reshape(B, NT, 128)
    tpad = jnp.pad(target_boxes, ((0, 0), (0, TG * 8 - T), (0, 0)))
    tm4, ti4 = _tc_call(cx4, cy4, w4, h4, tpad)
    tm = tm4[:, :, :, 0].reshape(B, TG * 8)[:, :T]
    ti = ti4[:, :, :, 0].reshape(B, TG * 8)[:, :T] + QTC0

    use_sc = sm >= tm
    max_iou = jnp.where(use_sc, sm, tm)
    pred_idx = jnp.where(use_sc, si, ti)
    return pred_idx, max_iou


def kernel(pred_boxes, target_boxes):
    pred_idx, max_iou = _matcher(pred_boxes, target_boxes)
    valid = max_iou >= 0.5
    return pred_idx, valid, max_iou


# speedup vs baseline: 7.6542x; 1.0565x over previous
import functools

import jax
import jax.numpy as jnp
from jax import lax
from jax.experimental import pallas as pl
from jax.experimental.pallas import tpu as pltpu
from jax.experimental.pallas import tpu_sc as plsc

B = 8
Q = 20000
T = 100
TPAD = 112
NCORE = 2
NSUB = 16
B_PER_CORE = B // NCORE
G = 4

QS = 9216
PER_W = 576
CHUNKS = PER_W // 16
NT = 88
QTC0 = Q - NT * 128
TC_U = 8
TG = 13
BIG = 1 << 30


def _sc_body(pred_hbm, tgt_hbm, outmax_hbm, outidx_hbm,
             praw8, x0a, y0a, x1a, y1a, aa, traw,
             lmax, lidx, mgmax, mgidx, omax, oidx,
             shmax, shidx):
    c = lax.axis_index("c")
    s = lax.axis_index("s")
    iota = lax.iota(jnp.int32, 16)
    lane0 = iota == 0
    zc = jnp.zeros((16,), jnp.int32)
    base = s * PER_W

    pltpu.sync_copy(pred_hbm.at[:, pl.ds(base * 4, PER_W * 4)], praw8)

    for bl in range(B_PER_CORE):
        b = c * B_PER_CORE + bl
        pltpu.sync_copy(tgt_hbm.at[b], traw)
        brow = zc + b

        def pre(j, _):
            r = j * 64 + iota * 4
            cx = plsc.load_gather(praw8, [brow, r])
            cy = plsc.load_gather(praw8, [brow, r + 1])
            w = plsc.load_gather(praw8, [brow, r + 2])
            h = plsc.load_gather(praw8, [brow, r + 3])
            x0 = cx - 0.5 * w
            y0 = cy - 0.5 * h
            x1 = cx + 0.5 * w
            y1 = cy + 0.5 * h
            sl = pl.ds(j * 16, 16)
            x0a[sl] = x0
            y0a[sl] = y0
            x1a[sl] = x1
            y1a[sl] = y1
            aa[sl] = (x1 - x0) * (y1 - y0)
            return 0

        lax.fori_loop(0, CHUNKS, pre, 0)

        def per_g(g, _):
            t0 = g * G
            tco = []
            for i in range(G):
                t4 = zc + (t0 + i) * 4
                tcx = plsc.load_gather(traw, [t4])
                tcy = plsc.load_gather(traw, [t4 + 1])
                tw = plsc.load_gather(traw, [t4 + 2])
                th = plsc.load_gather(traw, [t4 + 3])
                tx0 = tcx - 0.5 * tw
                ty0 = tcy - 0.5 * th
                tx1 = tcx + 0.5 * tw
                ty1 = tcy + 0.5 * th
                ta = (tx1 - tx0) * (ty1 - ty0)
                tco.append((tx0, ty0, tx1, ty1, ta))

            def scan_k(k, carry):
                ms, bis, idxv = carry
                sl = pl.ds(k * 16, 16)
                x0 = x0a[sl]
                y0 = y0a[sl]
                x1 = x1a[sl]
                y1 = y1a[sl]
                av = aa[sl]
                nms, nbis = [], []
                for i in range(G):
                    tx0, ty0, tx1, ty1, ta = tco[i]
                    ltx = jnp.maximum(x0, tx0)
                    lty = jnp.maximum(y0, ty0)
                    rbx = jnp.minimum(x1, tx1)
                    rby = jnp.minimum(y1, ty1)
                    inter = jnp.maximum(rbx - ltx, 0.0) * jnp.maximum(rby - lty, 0.0)
                    union = av + ta - inter
                    iou = inter / jnp.maximum(union, 1e-9)
                    lcx = jnp.minimum(x0, tx0)
                    lcy = jnp.minimum(y0, ty0)
                    rcx = jnp.maximum(x1, tx1)
                    rcy = jnp.maximum(y1, ty1)
                    areac = (rcx - lcx) * (rcy - lcy)
                    gv = iou - (areac - union) / jnp.maximum(areac, 1e-9)
                    upd = gv > ms[i]
                    nms.append(jnp.where(upd, gv, ms[i]))
                    nbis.append(jnp.where(upd, idxv, bis[i]))
                return tuple(nms), tuple(nbis), idxv + 16

            m0 = jnp.full((16,), -3.0e38, jnp.float32)
            bi0 = jnp.zeros((16,), jnp.int32)
            ms, bis, _ = lax.fori_loop(
                0, CHUNKS, scan_k,
                ((m0,) * G, (bi0,) * G, base + iota))
            for i in range(G):
                gm = jnp.max(ms[i])
                cand = jnp.where(ms[i] == jnp.full((16,), gm),
                                 bis[i], jnp.full((16,), BIG, jnp.int32))
                gi = jnp.min(cand)
                posv = zc + (bl * TPAD + t0 + i)
                plsc.store_scatter(lmax, [posv], jnp.full((16,), gm), mask=lane0)
                plsc.store_scatter(lidx, [posv], jnp.full((16,), gi, jnp.int32),
                                   mask=lane0)
            return 0

        lax.fori_loop(0, T // G, per_g, 0)

    pltpu.sync_copy(lmax, shmax.at[pl.ds(s * (B_PER_CORE * TPAD), B_PER_CORE * TPAD)])
    pltpu.sync_copy(lidx, shidx.at[pl.ds(s * (B_PER_CORE * TPAD), B_PER_CORE * TPAD)])
    plsc.subcore_barrier()

    @pl.when(s < B_PER_CORE)
    def _merge():
        for w in range(NSUB):
            pltpu.sync_copy(shmax.at[pl.ds(w * (B_PER_CORE * TPAD) + s * TPAD, TPAD)],
                            mgmax.at[pl.ds(w * TPAD, TPAD)])
            pltpu.sync_copy(shidx.at[pl.ds(w * (B_PER_CORE * TPAD) + s * TPAD, TPAD)],
                            mgidx.at[pl.ds(w * TPAD, TPAD)])

        def mg(cc, _):
            sl = pl.ds(cc * 16, 16)
            del _
            acc = mgmax[sl]
            acci = mgidx[sl]
            for w in range(1, NSUB):
                wsl = pl.ds(w * TPAD + cc * 16, 16)
                v = mgmax[wsl]
                vi = mgidx[wsl]
                upd = (v > acc) | ((v == acc) & (vi < acci))
                acc = jnp.where(upd, v, acc)
                acci = jnp.where(upd, vi, acci)
            omax[sl] = acc
            oidx[sl] = acci
            return 0

        lax.fori_loop(0, TPAD // 16, mg, 0)
        gb = c * B_PER_CORE + s
        pltpu.sync_copy(omax, outmax_hbm.at[pl.ds(gb * TPAD, TPAD)])
        pltpu.sync_copy(oidx, outidx_hbm.at[pl.ds(gb * TPAD, TPAD)])


def _tc_body(cxref, cyref, wref, href, tref, omax_ref, oidx_ref,
             x0s, y0s, x1s, y1s, aas):
    g = pl.program_id(1)

    @pl.when(g == 0)
    def _pre():
        cx = cxref[0]
        cy = cyref[0]
        w = wref[0]
        h = href[0]
        x0 = cx - 0.5 * w
        y0 = cy - 0.5 * h
        x1 = cx + 0.5 * w
        y1 = cy + 0.5 * h
        x0s[...] = x0
        y0s[...] = y0
        x1s[...] = x1
        y1s[...] = y1
        aas[...] = (x1 - x0) * (y1 - y0)

    tcx = tref[0, pl.ds(g * 8, 8), 0:1]
    tcy = tref[0, pl.ds(g * 8, 8), 1:2]
    tw = tref[0, pl.ds(g * 8, 8), 2:3]
    th = tref[0, pl.ds(g * 8, 8), 3:4]
    tx0 = jnp.broadcast_to(tcx - 0.5 * tw, (8, 128))
    ty0 = jnp.broadcast_to(tcy - 0.5 * th, (8, 128))
    tx1 = jnp.broadcast_to(tcx + 0.5 * tw, (8, 128))
    ty1 = jnp.broadcast_to(tcy + 0.5 * th, (8, 128))
    ta = (tx1 - tx0) * (ty1 - ty0)
    iota = lax.broadcasted_iota(jnp.int32, (8, 128), 1)

    def body(k, carry):
        ms, bis = carry
        nms, nbis = [], []
        for u in range(TC_U):
            kk = k * TC_U + u
            x0 = x0s[pl.ds(kk, 1), :]
            y0 = y0s[pl.ds(kk, 1), :]
            x1 = x1s[pl.ds(kk, 1), :]
            y1 = y1s[pl.ds(kk, 1), :]
            av = aas[pl.ds(kk, 1), :]
            ltx = jnp.maximum(x0, tx0)
            lty = jnp.maximum(y0, ty0)
            rbx = jnp.minimum(x1, tx1)
            rby = jnp.minimum(y1, ty1)
            inter = jnp.maximum(rbx - ltx, 0.0) * jnp.maximum(rby - lty, 0.0)
            union = av + ta - inter
            iou = inter / jnp.maximum(union, 1e-9)
            lcx = jnp.minimum(x0, tx0)
            lcy = jnp.minimum(y0, ty0)
            rcx = jnp.maximum(x1, tx1)
            rcy = jnp.maximum(y1, ty1)
            areac = (rcx - lcx) * (rcy - lcy)
            gv = iou - (areac - union) / jnp.maximum(areac, 1e-9)
            pidx = iota + kk * 128
            upd = gv > ms[u]
            nms.append(jnp.where(upd, gv, ms[u]))
            nbis.append(jnp.where(upd, pidx, bis[u]))
        return tuple(nms), tuple(nbis)

    m0 = jnp.full((8, 128), -3.0e38, jnp.float32)
    bi0 = jnp.zeros((8, 128), jnp.int32)
    ms, bis = lax.fori_loop(0, NT // TC_U, body,
                            ((m0,) * TC_U, (bi0,) * TC_U))
    m, bi = ms[0], bis[0]
    for u in range(1, TC_U):
        upd = (ms[u] > m) | ((ms[u] == m) & (bis[u] < bi))
        m = jnp.where(upd, ms[u], m)
        bi = jnp.where(upd, bis[u], bi)
    gm = jnp.max(m, axis=1, keepdims=True)
    cand = jnp.where(m == gm, bi, BIG)
    gi = jnp.min(cand, axis=1, keepdims=True)
    omax_ref[0, 0] = jnp.broadcast_to(gm, (8, 128))
    oidx_ref[0, 0] = jnp.broadcast_to(gi, (8, 128))


_tc_call = pl.pallas_call(
    _tc_body,
    grid=(B, TG),
    in_specs=[
        pl.BlockSpec((1, NT, 128), lambda i, g: (i, 0, 0)),
        pl.BlockSpec((1, NT, 128), lambda i, g: (i, 0, 0)),
        pl.BlockSpec((1, NT, 128), lambda i, g: (i, 0, 0)),
        pl.BlockSpec((1, NT, 128), lambda i, g: (i, 0, 0)),
        pl.BlockSpec((1, TG * 8, 4), lambda i, g: (i, 0, 0)),
    ],
    out_specs=[
        pl.BlockSpec((1, 1, 8, 128), lambda i, g: (i, g, 0, 0)),
        pl.BlockSpec((1, 1, 8, 128), lambda i, g: (i, g, 0, 0)),
    ],
    out_shape=[
        jax.ShapeDtypeStruct((B, TG, 8, 128), jnp.float32),
        jax.ShapeDtypeStruct((B, TG, 8, 128), jnp.int32),
    ],
    scratch_shapes=[pltpu.VMEM((NT, 128), jnp.float32) for _ in range(5)],
)


@jax.jit
def _matcher(pred_boxes, target_boxes):
    sc = pl.kernel(
        _sc_body,
        out_type=[
            jax.ShapeDtypeStruct((B * TPAD,), jnp.float32),
            jax.ShapeDtypeStruct((B * TPAD,), jnp.int32),
        ],
        mesh=plsc.VectorSubcoreMesh(core_axis_name="c", subcore_axis_name="s",
                                    num_cores=NCORE, num_subcores=NSUB),
        compiler_params=pltpu.CompilerParams(needs_layout_passes=False),
        scratch_types=[
            pltpu.VMEM((B, PER_W * 4), jnp.float32),
            pltpu.VMEM((PER_W,), jnp.float32),
            pltpu.VMEM((PER_W,), jnp.float32),
            pltpu.VMEM((PER_W,), jnp.float32),
            pltpu.VMEM((PER_W,), jnp.float32),
            pltpu.VMEM((PER_W,), jnp.float32),
            pltpu.VMEM((T * 4,), jnp.float32),
            pltpu.VMEM((B_PER_CORE * TPAD,), jnp.float32),
            pltpu.VMEM((B_PER_CORE * TPAD,), jnp.int32),
            pltpu.VMEM((NSUB * TPAD,), jnp.float32),
            pltpu.VMEM((NSUB * TPAD,), jnp.int32),
            pltpu.VMEM((TPAD,), jnp.float32),
            pltpu.VMEM((TPAD,), jnp.int32),
            pltpu.VMEM_SHARED((NSUB * B_PER_CORE * TPAD,), jnp.float32),
            pltpu.VMEM_SHARED((NSUB * B_PER_CORE * TPAD,), jnp.int32),
        ],
    )
    sm, si = sc(pred_boxes[:, :QS, :].reshape(B, QS * 4),
                target_boxes.reshape(B, T * 4))
    sm = sm.reshape(B, TPAD)[:, :T]
    si = si.reshape(B, TPAD)[:, :T]

    ptc = pred_boxes[:, QTC0:, :]
    cx4 = ptc[:, :, 0].reshape(B, NT, 128)
    cy4 = ptc[:, :, 1].reshape(B, NT, 128)
    w4 = ptc[:, :, 2].reshape(B, NT, 128)
    h4 = ptc[:, :, 3].reshape(B, NT, 128)
    tpad = jnp.pad(target_boxes, ((0, 0), (0, TG * 8 - T), (0, 0)))
    tm4, ti4 = _tc_call(cx4, cy4, w4, h4, tpad)
    tm = tm4[:, :, :, 0].reshape(B, TG * 8)[:, :T]
    ti = ti4[:, :, :, 0].reshape(B, TG * 8)[:, :T] + QTC0

    use_sc = sm >= tm
    max_iou = jnp.where(use_sc, sm, tm)
    pred_idx = jnp.where(use_sc, si, ti)
    return pred_idx, max_iou


def kernel(pred_boxes, target_boxes):
    pred_idx, max_iou = _matcher(pred_boxes, target_boxes)
    valid = max_iou >= 0.5
    return pred_idx, valid, max_iou
